# baseline (speedup 1.0000x reference)
import functools

import jax
import jax.numpy as jnp
from jax import lax
from jax.experimental import pallas as pl
from jax.experimental.pallas import tpu as pltpu
from jax.experimental.pallas import tpu_sc as plsc

N, K, C = 1024, 64, 512
V, P, M = 256, 256, 32768
NK = N * K
NC, NS = 2, 16
NW = NC * NS
CH = 128
BPW = M // NW
SCALE = float(C) ** -0.5
_mesh = functools.partial(plsc.VectorSubcoreMesh,
                          core_axis_name="c", subcore_axis_name="s")


def _gelu_exact(x):
    z = x * (2.0 ** -0.5)
    az = jnp.abs(z)
    t = 1.0 / (1.0 + 0.3275911 * az)
    poly = t * (0.254829592 + t * (-0.284496736 + t * (1.421413741
               + t * (-1.453152027 + t * 1.061405429))))
    erf_az = 1.0 - poly * jnp.exp(-az * az)
    erf_z = jnp.sign(z) * erf_az
    return x * 0.5 * (1.0 + erf_z)


def _ln(x, g, b):
    mu = jnp.mean(x, axis=-1, keepdims=True)
    var = jnp.mean((x - mu) ** 2, axis=-1, keepdims=True)
    return (x - mu) * lax.rsqrt(var + 1e-5) * g + b


def _point_body(x, WpP, bpP, gpP, bepP, Wpub, bpub, gpub, bepub,
                gca, bca, Wq, Wk, Wv, bctx,
                pub_o, pri_o, q_o, qbf_o, kv_o):
    xv = x[...]
    pri = jax.nn.relu(_ln(xv @ WpP[...] + bpP[...], gpP[...], bepP[...]))
    pub = jax.nn.relu(_ln(xv @ Wpub[...] + bpub[...], gpub[...], bepub[...]))
    xn = _ln(pub, gca[...], bca[...])
    q = xn @ Wq[...]
    q_o[...] = q
    qbf_o[...] = q.astype(jnp.bfloat16)
    pub_o[...] = pub
    pri_o[...] = pri
    kv_o[...] = jnp.concatenate([bctx[...] @ Wk[...], bctx[...] @ Wv[...]], 0)


def _point_call(x, p):
    r = lambda v: v.reshape(1, -1)
    return pl.pallas_call(
        _point_body,
        out_shape=[jax.ShapeDtypeStruct((N, C), jnp.float32),
                   jax.ShapeDtypeStruct((N, C), jnp.float32),
                   jax.ShapeDtypeStruct((N, C), jnp.float32),
                   jax.ShapeDtypeStruct((N, C), jnp.bfloat16),
                   jax.ShapeDtypeStruct((2, C), jnp.float32)],
    )(x, p['W_pP'], r(p['b_pP']), r(p['g_pP']), r(p['be_pP']),
      p['W_pub'], r(p['b_pub']), r(p['g_pub']), r(p['be_pub']),
      r(p['g_ca']), r(p['b_ca']), p['Wq'], p['Wk'], p['Wv'], r(p['b_ctx']))


def _sc_gather_rows2(table1, idx1, table2, idx2):
    B = idx1.shape[0]
    D = table1.shape[1]
    bpw = B // NW
    cg = 64
    nch = bpw // cg

    @functools.partial(
        pl.kernel,
        out_type=[jax.ShapeDtypeStruct((B, D), jnp.float32),
                  jax.ShapeDtypeStruct((B, D), jnp.float32)],
        mesh=_mesh(),
        scratch_types=[pltpu.VMEM((cg,), jnp.int32),
                       pltpu.VMEM((cg,), jnp.int32),
                       pltpu.VMEM((cg, D), jnp.float32),
                       pltpu.VMEM((cg, D), jnp.float32),
                       pltpu.SemaphoreType.DMA,
                       pltpu.SemaphoreType.DMA,
                       pltpu.SemaphoreType.DMA],
    )
    def k(t1_hbm, i1_hbm, t2_hbm, i2_hbm, o1_hbm, o2_hbm,
          ix1, ix2, r1, r2, semx, semq, sems):
        wid = lax.axis_index("s") * NC + lax.axis_index("c")
        base = wid * bpw

        def body(i, carry):
            off = pl.multiple_of(base + i * cg, cg)
            pltpu.sync_copy(i1_hbm.at[pl.ds(off, cg)], ix1)
            pltpu.sync_copy(i2_hbm.at[pl.ds(off, cg)], ix2)
            c1 = pltpu.async_copy(t1_hbm.at[ix1], r1, semx)
            c2 = pltpu.async_copy(t2_hbm.at[ix2], r2, semq)
            c1.wait()
            s1 = pltpu.async_copy(r1, o1_hbm.at[pl.ds(off, cg)], sems)
            c2.wait()
            s2 = pltpu.async_copy(r2, o2_hbm.at[pl.ds(off, cg)], sems)
            s1.wait()
            s2.wait()
            return carry

        lax.fori_loop(0, nch, body, 0)

    return k(table1, idx1, table2, idx2)


def _sc_gather_flat(table, idx):
    B = idx.shape[0]
    bpw = B // NW
    nch = bpw // CH

    @functools.partial(
        pl.kernel,
        out_type=jax.ShapeDtypeStruct((B,), jnp.float32),
        mesh=_mesh(),
        scratch_types=[pltpu.VMEM((CH,), jnp.int32),
                       pltpu.VMEM((CH,), jnp.float32),
                       pltpu.SemaphoreType.DMA],
    )
    def k(table_hbm, idx_hbm, out_hbm, idx_v, val_v, sem):
        wid = lax.axis_index("s") * NC + lax.axis_index("c")
        base = wid * bpw

        def body(i, carry):
            off = pl.multiple_of(base + i * CH, CH)
            pltpu.sync_copy(idx_hbm.at[pl.ds(off, CH)], idx_v)
            pltpu.async_copy(table_hbm.at[idx_v], val_v, sem).wait()
            pltpu.sync_copy(val_v, out_hbm.at[pl.ds(off, CH)])
            return carry

        lax.fori_loop(0, nch, body, 0)

    return k(table, idx)


def _rows_body(X, QG, Wpub, bpub, gpub, bepub, WpI, bpI, gpI, bepI,
               gctx, bctx, Wk, Wv, pub_o, pri_o, vr_o, s_o):
    x = X[...]
    pub = jax.nn.relu(_ln(x @ Wpub[...] + bpub[...], gpub[...], bepub[...]))
    pri = jax.nn.relu(_ln(x @ WpI[...] + bpI[...], gpI[...], bepI[...]))
    kc = _ln(pub, gctx[...], bctx[...])
    vc = _ln(pri, gctx[...], bctx[...])
    kr = kc @ Wk[...]
    vr = vc @ Wv[...]
    s = jnp.sum(kr * QG[...].astype(jnp.float32), axis=1) * SCALE
    pub_o[...] = pub
    pri_o[...] = pri
    vr_o[...] = vr
    s_o[...] = (s + 1024.0)[None, None, :]


def _rows_call(X, QG, p):
    BR = 512
    G = M // BR
    r = lambda v: v.reshape(1, -1)
    row = pl.BlockSpec((BR, C), lambda g: (g, 0))
    full = lambda a, b: pl.BlockSpec((a, b), lambda g: (0, 0))
    return pl.pallas_call(
        _rows_body,
        grid=(G,),
        in_specs=[row, row,
                  full(C, C), full(1, C), full(1, C), full(1, C),
                  full(C, C), full(1, C), full(1, C), full(1, C),
                  full(1, C), full(1, C), full(C, C), full(C, C)],
        out_specs=[row, row, row, pl.BlockSpec((1, 1, BR), lambda g: (g, 0, 0))],
        out_shape=[jax.ShapeDtypeStruct((M, C), jnp.float32),
                   jax.ShapeDtypeStruct((M, C), jnp.float32),
                   jax.ShapeDtypeStruct((M, C), jnp.float32),
                   jax.ShapeDtypeStruct((G, 1, BR), jnp.float32)],
    )(X, QG, p['W_pub'], r(p['b_pub']), r(p['g_pub']), r(p['be_pub']),
      p['W_pI'], r(p['b_pI']), r(p['g_pI']), r(p['be_pI']),
      r(p['g_ctx']), r(p['b_ctx']), p['Wk'], p['Wv'])


def _sc_scatter_scores(s_flat, idx_adj):
    nch = BPW // CH
    zch = NK // NS // CH

    @functools.partial(
        pl.kernel,
        out_type=jax.ShapeDtypeStruct((NC * NK,), jnp.float32),
        mesh=_mesh(),
        scratch_types=[pltpu.VMEM((CH,), jnp.int32),
                       pltpu.VMEM((CH,), jnp.float32),
                       pltpu.VMEM((CH,), jnp.float32)],
    )
    def k(s_hbm, idx_hbm, sd_hbm, idx_v, val_v, zero_v):
        cid = lax.axis_index("c")
        sid = lax.axis_index("s")
        wid = sid * NC + cid
        for i in range(CH // 16):
            zero_v[pl.ds(i * 16, 16)] = jnp.zeros((16,), jnp.float32)
        zbase = cid * NK + sid * (NK // NS)
        for j in range(zch):
            dst = pl.ds(pl.multiple_of(zbase + j * CH, CH), CH)
            pltpu.sync_copy(zero_v, sd_hbm.at[dst])
        plsc.subcore_barrier()
        base = wid * BPW

        def body(i, carry):
            off = pl.multiple_of(base + i * CH, CH)
            pltpu.sync_copy(idx_hbm.at[pl.ds(off, CH)], idx_v)
            pltpu.sync_copy(s_hbm.at[pl.ds(off, CH)], val_v)
            pltpu.sync_copy(val_v, sd_hbm.at[idx_v])
            return carry

        lax.fori_loop(0, nch, body, 0)

    return k(s_flat, idx_adj)


def _softmax_body(sd, q, kv, attn_o, aux_o):
    k_const = kv[0, :]
    s0 = jnp.sum(q[...] * k_const[None, :], axis=1) * SCALE
    sv = sd[0] + sd[1]
    mk = (sv > 512.0).astype(jnp.float32)
    S = jnp.where(mk > 0, sv - 1024.0, s0[:, None])
    mx = jnp.max(S, axis=1, keepdims=True)
    e = jnp.exp(S - mx)
    Z = jnp.sum(e, axis=1, keepdims=True)
    attn_o[...] = e / Z
    w0 = jnp.sum(e * (1.0 - mk), axis=1) / Z[:, 0]
    cnt = jnp.sum(mk, axis=1)
    aux_o[...] = jnp.concatenate(
        [w0[:, None], cnt[:, None], jnp.zeros((w0.shape[0], 6), jnp.float32)], 1)


def _softmax_call(sd, q, kv):
    BN = 256
    G = N // BN
    return pl.pallas_call(
        _softmax_body,
        grid=(G,),
        in_specs=[pl.BlockSpec((NC, BN, K), lambda g: (0, g, 0)),
                  pl.BlockSpec((BN, C), lambda g: (g, 0)),
                  pl.BlockSpec((2, C), lambda g: (0, 0))],
        out_specs=[pl.BlockSpec((BN, K), lambda g: (g, 0)),
                   pl.BlockSpec((BN, 8), lambda g: (g, 0))],
        out_shape=[jax.ShapeDtypeStruct((N, K), jnp.float32),
                   jax.ShapeDtypeStruct((N, 8), jnp.float32)],
    )(sd, q, kv)


def _seg_body(pub, pri, vr, a3, nid3, out_ref):
    g = pl.program_id(0)

    @pl.when(g == 0)
    def _():
        out_ref[...] = jnp.zeros_like(out_ref)

    nid = nid3[0, 0, :]
    BR = nid.shape[0]
    onehot = (nid[:, None] ==
              lax.broadcasted_iota(jnp.int32, (BR, N), 1)).astype(jnp.float32)
    a = a3[0, 0, :]
    avr = vr[...] * a[:, None]
    dn = (((0,), (0,)), ((), ()))
    out_ref[0] += lax.dot_general(onehot, pub[...], dn,
                                  preferred_element_type=jnp.float32)
    out_ref[1] += lax.dot_general(onehot, pri[...], dn,
                                  preferred_element_type=jnp.float32)
    out_ref[2] += lax.dot_general(onehot, avr, dn,
                                  preferred_element_type=jnp.float32)


def _seg_call(pub, pri, vr, a3, nid3):
    BR = 512
    G = M // BR
    row = pl.BlockSpec((BR, C), lambda g: (g, 0))
    e3 = pl.BlockSpec((1, 1, BR), lambda g: (g, 0, 0))
    return pl.pallas_call(
        _seg_body,
        grid=(G,),
        in_specs=[row, row, row, e3, e3],
        out_specs=pl.BlockSpec((3, N, C), lambda g: (0, 0, 0)),
        out_shape=jax.ShapeDtypeStruct((3, N, C), jnp.float32),
    )(pub, pri, vr, a3, nid3)


def _final_body(parts, pubp, prip, aux, kv,
                Wo, bo, gff, bff, W1, b1, W2, b2,
                Wfp_a, Wfp_b, bfp, Wfpr_a, Wfpr_b, bfpr,
                feat_o, poolpub_o, poolpri_o):
    pubsum = parts[0]
    prisum = parts[1]
    attnsum = parts[2]
    w0 = aux[:, 0]
    cnt = aux[:, 1]
    v_const = kv[1, :]
    pq = pubp[...]
    attn_out = attnsum + w0[:, None] * v_const[None, :]
    out = attn_out @ Wo[...] + bo[...]
    pp1 = out + pq
    h = _ln(pp1, gff[...], bff[...]) @ W1[...] + b1[...]
    a = h[:, :4 * C]
    gch = h[:, 4 * C:]
    h2 = (a * _gelu_exact(gch)) @ W2[...] + b2[...]
    pfused = h2 + pp1
    fpub = jax.nn.sigmoid(pq @ Wfp_a[...] + pfused @ Wfp_b[...] + bfp[...])
    feat_o[...] = jax.nn.sigmoid(
        prip[...] @ Wfpr_a[...] + fpub @ Wfpr_b[...] + bfpr[...])
    inv_k = 1.0 / K
    poolpub_o[...] = (pubsum + (K - cnt)[:, None]) * inv_k
    poolpri_o[...] = (prisum + (K - cnt)[:, None]) * inv_k


def _final_call(parts, pubp, prip, aux, kv, p):
    BN = 256
    G = N // BN
    r = lambda v: v.reshape(1, -1)
    row = pl.BlockSpec((BN, C), lambda g: (g, 0))
    full = lambda a, b: pl.BlockSpec((a, b), lambda g: (0, 0))
    return pl.pallas_call(
        _final_body,
        grid=(G,),
        in_specs=[pl.BlockSpec((3, BN, C), lambda g: (0, g, 0)),
                  row, row, pl.BlockSpec((BN, 8), lambda g: (g, 0)),
                  full(2, C),
                  full(C, C), full(1, C), full(1, C), full(1, C),
                  full(C, 8 * C), full(1, 8 * C),
                  full(4 * C, C), full(1, C),
                  full(C, C), full(C, C), full(1, C),
                  full(C, C), full(C, C), full(1, C)],
        out_specs=[row, row, row],
        out_shape=[jax.ShapeDtypeStruct((N, C), jnp.float32),
                   jax.ShapeDtypeStruct((N, C), jnp.float32),
                   jax.ShapeDtypeStruct((N, C), jnp.float32)],
    )(parts, pubp, prip, aux, kv,
      p['Wo'], r(p['bo']), r(p['g_ff']), r(p['b_ff']),
      p['W1'], r(p['b1']), p['W2'], r(p['b2']),
      p['W_fp'][:C], p['W_fp'][C:], r(p['b_fp']),
      p['W_fpr'][:C], p['W_fpr'][C:], r(p['b_fpr']))


def kernel(image_feats, point_feats, mask, inds2d, inds3d, params):
    p = params
    flat2d = inds2d[:, 0] * P + inds2d[:, 1]
    n_ids = inds3d[:, 0]
    flat3d = n_ids * K + inds3d[:, 1]
    core_of = (jnp.arange(M, dtype=jnp.int32) // BPW) % NC
    flat3d_adj = flat3d + core_of * NK

    pub_p, pri_p, q, q_bf, kv = _point_call(point_feats[0], p)
    X, QG = _sc_gather_rows2(image_feats.reshape(V * P, C), flat2d, q, n_ids)
    pub, pri, vr, s3 = _rows_call(X, QG, p)
    sd = _sc_scatter_scores(s3.reshape(M), flat3d_adj)
    attn, aux = _softmax_call(sd.reshape(NC, N, K), q, kv)
    a_elem = _sc_gather_flat(attn.reshape(NK), flat3d)
    parts = _seg_call(pub, pri, vr, a_elem.reshape(M // 512, 1, 512),
                      n_ids.reshape(M // 512, 1, 512))
    feat, poolpub, poolpri = _final_call(parts, pub_p, pri_p, aux, kv, p)
    return feat, pub_p, pri_p, poolpub, poolpri


# speedup vs baseline: 1.0224x; 1.0224x over previous
import functools

import jax
import jax.numpy as jnp
from jax import lax
from jax.experimental import pallas as pl
from jax.experimental.pallas import tpu as pltpu
from jax.experimental.pallas import tpu_sc as plsc

N, K, C = 1024, 64, 512
V, P, M = 256, 256, 32768
NK = N * K
NC, NS = 2, 16
NW = NC * NS
CH = 128
BPW = M // NW
SCALE = float(C) ** -0.5
_mesh = functools.partial(plsc.VectorSubcoreMesh,
                          core_axis_name="c", subcore_axis_name="s")


def _gelu_exact(x):
    z = x * (2.0 ** -0.5)
    az = jnp.abs(z)
    t = 1.0 / (1.0 + 0.3275911 * az)
    poly = t * (0.254829592 + t * (-0.284496736 + t * (1.421413741
               + t * (-1.453152027 + t * 1.061405429))))
    erf_az = 1.0 - poly * jnp.exp(-az * az)
    erf_z = jnp.sign(z) * erf_az
    return x * 0.5 * (1.0 + erf_z)


def _ln(x, g, b):
    mu = jnp.mean(x, axis=-1, keepdims=True)
    var = jnp.mean((x - mu) ** 2, axis=-1, keepdims=True)
    return (x - mu) * lax.rsqrt(var + 1e-5) * g + b


def _point_body(x, WpP, bpP, gpP, bepP, Wpub, bpub, gpub, bepub,
                gca, bca, Wq, Wk, Wv, bctx,
                pub_o, pri_o, q_o, qbf_o, kv_o):
    xv = x[...]
    pri = jax.nn.relu(_ln(xv @ WpP[...] + bpP[...], gpP[...], bepP[...]))
    pub = jax.nn.relu(_ln(xv @ Wpub[...] + bpub[...], gpub[...], bepub[...]))
    xn = _ln(pub, gca[...], bca[...])
    q = xn @ Wq[...]
    q_o[...] = q
    qbf_o[...] = q.astype(jnp.bfloat16)
    pub_o[...] = pub
    pri_o[...] = pri
    kv_o[...] = jnp.concatenate([bctx[...] @ Wk[...], bctx[...] @ Wv[...]], 0)


def _point_call(x, p):
    r = lambda v: v.reshape(1, -1)
    return pl.pallas_call(
        _point_body,
        out_shape=[jax.ShapeDtypeStruct((N, C), jnp.float32),
                   jax.ShapeDtypeStruct((N, C), jnp.float32),
                   jax.ShapeDtypeStruct((N, C), jnp.float32),
                   jax.ShapeDtypeStruct((N, C), jnp.bfloat16),
                   jax.ShapeDtypeStruct((2, C), jnp.float32)],
    )(x, p['W_pP'], r(p['b_pP']), r(p['g_pP']), r(p['be_pP']),
      p['W_pub'], r(p['b_pub']), r(p['g_pub']), r(p['be_pub']),
      r(p['g_ca']), r(p['b_ca']), p['Wq'], p['Wk'], p['Wv'], r(p['b_ctx']))


def _sc_gather_rows2(table1, idx1, table2, idx2):
    B = idx1.shape[0]
    D = table1.shape[1]
    bpw = B // NW
    nch = bpw // CH

    @functools.partial(
        pl.kernel,
        out_type=[jax.ShapeDtypeStruct((B, D), jnp.float32),
                  jax.ShapeDtypeStruct((B, D), jnp.float32)],
        mesh=_mesh(),
        scratch_types=[pltpu.VMEM((CH,), jnp.int32),
                       pltpu.VMEM((CH, D), jnp.float32),
                       pltpu.SemaphoreType.DMA],
    )
    def k(t1_hbm, i1_hbm, t2_hbm, i2_hbm, o1_hbm, o2_hbm, idx_v, rows_v, sem):
        wid = lax.axis_index("s") * NC + lax.axis_index("c")
        base = wid * bpw

        def mk_body(t_hbm, i_hbm, o_hbm):
            def body(i, carry):
                off = pl.multiple_of(base + i * CH, CH)
                pltpu.sync_copy(i_hbm.at[pl.ds(off, CH)], idx_v)
                pltpu.async_copy(t_hbm.at[idx_v], rows_v, sem).wait()
                pltpu.sync_copy(rows_v, o_hbm.at[pl.ds(off, CH)])
                return carry
            return body

        lax.fori_loop(0, nch, mk_body(t1_hbm, i1_hbm, o1_hbm), 0)
        lax.fori_loop(0, nch, mk_body(t2_hbm, i2_hbm, o2_hbm), 0)

    return k(table1, idx1, table2, idx2)


def _sc_gather_flat(table, idx):
    B = idx.shape[0]
    bpw = B // NW
    nch = bpw // CH

    @functools.partial(
        pl.kernel,
        out_type=jax.ShapeDtypeStruct((B,), jnp.float32),
        mesh=_mesh(),
        scratch_types=[pltpu.VMEM((CH,), jnp.int32),
                       pltpu.VMEM((CH,), jnp.float32),
                       pltpu.SemaphoreType.DMA],
    )
    def k(table_hbm, idx_hbm, out_hbm, idx_v, val_v, sem):
        wid = lax.axis_index("s") * NC + lax.axis_index("c")
        base = wid * bpw

        def body(i, carry):
            off = pl.multiple_of(base + i * CH, CH)
            pltpu.sync_copy(idx_hbm.at[pl.ds(off, CH)], idx_v)
            pltpu.async_copy(table_hbm.at[idx_v], val_v, sem).wait()
            pltpu.sync_copy(val_v, out_hbm.at[pl.ds(off, CH)])
            return carry

        lax.fori_loop(0, nch, body, 0)

    return k(table, idx)


def _rows_body(X, QG, Wpub, bpub, gpub, bepub, WpI, bpI, gpI, bepI,
               gctx, bctx, Wk, Wv, pub_o, pri_o, vr_o, s_o):
    x = X[...]
    pub = jax.nn.relu(_ln(x @ Wpub[...] + bpub[...], gpub[...], bepub[...]))
    pri = jax.nn.relu(_ln(x @ WpI[...] + bpI[...], gpI[...], bepI[...]))
    kc = _ln(pub, gctx[...], bctx[...])
    vc = _ln(pri, gctx[...], bctx[...])
    kr = kc @ Wk[...]
    vr = vc @ Wv[...]
    s = jnp.sum(kr * QG[...].astype(jnp.float32), axis=1) * SCALE
    pub_o[...] = pub
    pri_o[...] = pri
    vr_o[...] = vr
    s_o[...] = (s + 1024.0)[None, None, :]


def _rows_call(X, QG, p):
    BR = 1024
    G = M // BR
    r = lambda v: v.reshape(1, -1)
    row = pl.BlockSpec((BR, C), lambda g: (g, 0))
    full = lambda a, b: pl.BlockSpec((a, b), lambda g: (0, 0))
    return pl.pallas_call(
        _rows_body,
        grid=(G,),
        in_specs=[row, row,
                  full(C, C), full(1, C), full(1, C), full(1, C),
                  full(C, C), full(1, C), full(1, C), full(1, C),
                  full(1, C), full(1, C), full(C, C), full(C, C)],
        out_specs=[row, row, row, pl.BlockSpec((1, 1, BR), lambda g: (g, 0, 0))],
        out_shape=[jax.ShapeDtypeStruct((M, C), jnp.float32),
                   jax.ShapeDtypeStruct((M, C), jnp.float32),
                   jax.ShapeDtypeStruct((M, C), jnp.float32),
                   jax.ShapeDtypeStruct((G, 1, BR), jnp.float32)],
    )(X, QG, p['W_pub'], r(p['b_pub']), r(p['g_pub']), r(p['be_pub']),
      p['W_pI'], r(p['b_pI']), r(p['g_pI']), r(p['be_pI']),
      r(p['g_ctx']), r(p['b_ctx']), p['Wk'], p['Wv'])


def _sc_scatter_scores(s_flat, idx_adj):
    nch = BPW // CH
    zch = NK // NS // CH

    @functools.partial(
        pl.kernel,
        out_type=jax.ShapeDtypeStruct((NC * NK,), jnp.float32),
        mesh=_mesh(),
        scratch_types=[pltpu.VMEM((CH,), jnp.int32),
                       pltpu.VMEM((CH,), jnp.float32),
                       pltpu.VMEM((CH,), jnp.float32)],
    )
    def k(s_hbm, idx_hbm, sd_hbm, idx_v, val_v, zero_v):
        cid = lax.axis_index("c")
        sid = lax.axis_index("s")
        wid = sid * NC + cid
        for i in range(CH // 16):
            zero_v[pl.ds(i * 16, 16)] = jnp.zeros((16,), jnp.float32)
        zbase = cid * NK + sid * (NK // NS)
        for j in range(zch):
            dst = pl.ds(pl.multiple_of(zbase + j * CH, CH), CH)
            pltpu.sync_copy(zero_v, sd_hbm.at[dst])
        plsc.subcore_barrier()
        base = wid * BPW

        def body(i, carry):
            off = pl.multiple_of(base + i * CH, CH)
            pltpu.sync_copy(idx_hbm.at[pl.ds(off, CH)], idx_v)
            pltpu.sync_copy(s_hbm.at[pl.ds(off, CH)], val_v)
            pltpu.sync_copy(val_v, sd_hbm.at[idx_v])
            return carry

        lax.fori_loop(0, nch, body, 0)

    return k(s_flat, idx_adj)


def _softmax_body(sd, q, kv, attn_o, aux_o):
    k_const = kv[0, :]
    s0 = jnp.sum(q[...] * k_const[None, :], axis=1) * SCALE
    sv = sd[0] + sd[1]
    mk = (sv > 512.0).astype(jnp.float32)
    S = jnp.where(mk > 0, sv - 1024.0, s0[:, None])
    mx = jnp.max(S, axis=1, keepdims=True)
    e = jnp.exp(S - mx)
    Z = jnp.sum(e, axis=1, keepdims=True)
    attn_o[...] = e / Z
    w0 = jnp.sum(e * (1.0 - mk), axis=1) / Z[:, 0]
    cnt = jnp.sum(mk, axis=1)
    aux_o[...] = jnp.concatenate(
        [w0[:, None], cnt[:, None], jnp.zeros((w0.shape[0], 6), jnp.float32)], 1)


def _softmax_call(sd, q, kv):
    BN = 256
    G = N // BN
    return pl.pallas_call(
        _softmax_body,
        grid=(G,),
        in_specs=[pl.BlockSpec((NC, BN, K), lambda g: (0, g, 0)),
                  pl.BlockSpec((BN, C), lambda g: (g, 0)),
                  pl.BlockSpec((2, C), lambda g: (0, 0))],
        out_specs=[pl.BlockSpec((BN, K), lambda g: (g, 0)),
                   pl.BlockSpec((BN, 8), lambda g: (g, 0))],
        out_shape=[jax.ShapeDtypeStruct((N, K), jnp.float32),
                   jax.ShapeDtypeStruct((N, 8), jnp.float32)],
    )(sd, q, kv)


def _seg_body(pub, pri, vr, a3, nid3, out_ref):
    g = pl.program_id(0)

    @pl.when(g == 0)
    def _():
        out_ref[...] = jnp.zeros_like(out_ref)

    nid = nid3[0, 0, :]
    BR = nid.shape[0]
    onehot = (nid[:, None] ==
              lax.broadcasted_iota(jnp.int32, (BR, N), 1)).astype(jnp.float32)
    a = a3[0, 0, :]
    avr = vr[...] * a[:, None]
    dn = (((0,), (0,)), ((), ()))
    out_ref[0] += lax.dot_general(onehot, pub[...], dn,
                                  preferred_element_type=jnp.float32)
    out_ref[1] += lax.dot_general(onehot, pri[...], dn,
                                  preferred_element_type=jnp.float32)
    out_ref[2] += lax.dot_general(onehot, avr, dn,
                                  preferred_element_type=jnp.float32)


def _seg_call(pub, pri, vr, a3, nid3):
    BR = 1024
    G = M // BR
    row = pl.BlockSpec((BR, C), lambda g: (g, 0))
    e3 = pl.BlockSpec((1, 1, BR), lambda g: (g, 0, 0))
    return pl.pallas_call(
        _seg_body,
        grid=(G,),
        in_specs=[row, row, row, e3, e3],
        out_specs=pl.BlockSpec((3, N, C), lambda g: (0, 0, 0)),
        out_shape=jax.ShapeDtypeStruct((3, N, C), jnp.float32),
    )(pub, pri, vr, a3, nid3)


def _final_body(parts, pubp, prip, aux, kv,
                Wo, bo, gff, bff, W1, b1, W2, b2,
                Wfp_a, Wfp_b, bfp, Wfpr_a, Wfpr_b, bfpr,
                feat_o, poolpub_o, poolpri_o):
    pubsum = parts[0]
    prisum = parts[1]
    attnsum = parts[2]
    w0 = aux[:, 0]
    cnt = aux[:, 1]
    v_const = kv[1, :]
    pq = pubp[...]
    attn_out = attnsum + w0[:, None] * v_const[None, :]
    out = attn_out @ Wo[...] + bo[...]
    pp1 = out + pq
    h = _ln(pp1, gff[...], bff[...]) @ W1[...] + b1[...]
    a = h[:, :4 * C]
    gch = h[:, 4 * C:]
    h2 = (a * _gelu_exact(gch)) @ W2[...] + b2[...]
    pfused = h2 + pp1
    fpub = jax.nn.sigmoid(pq @ Wfp_a[...] + pfused @ Wfp_b[...] + bfp[...])
    feat_o[...] = jax.nn.sigmoid(
        prip[...] @ Wfpr_a[...] + fpub @ Wfpr_b[...] + bfpr[...])
    inv_k = 1.0 / K
    poolpub_o[...] = (pubsum + (K - cnt)[:, None]) * inv_k
    poolpri_o[...] = (prisum + (K - cnt)[:, None]) * inv_k


def _final_call(parts, pubp, prip, aux, kv, p):
    BN = 256
    G = N // BN
    r = lambda v: v.reshape(1, -1)
    row = pl.BlockSpec((BN, C), lambda g: (g, 0))
    full = lambda a, b: pl.BlockSpec((a, b), lambda g: (0, 0))
    return pl.pallas_call(
        _final_body,
        grid=(G,),
        in_specs=[pl.BlockSpec((3, BN, C), lambda g: (0, g, 0)),
                  row, row, pl.BlockSpec((BN, 8), lambda g: (g, 0)),
                  full(2, C),
                  full(C, C), full(1, C), full(1, C), full(1, C),
                  full(C, 8 * C), full(1, 8 * C),
                  full(4 * C, C), full(1, C),
                  full(C, C), full(C, C), full(1, C),
                  full(C, C), full(C, C), full(1, C)],
        out_specs=[row, row, row],
        out_shape=[jax.ShapeDtypeStruct((N, C), jnp.float32),
                   jax.ShapeDtypeStruct((N, C), jnp.float32),
                   jax.ShapeDtypeStruct((N, C), jnp.float32)],
    )(parts, pubp, prip, aux, kv,
      p['Wo'], r(p['bo']), r(p['g_ff']), r(p['b_ff']),
      p['W1'], r(p['b1']), p['W2'], r(p['b2']),
      p['W_fp'][:C], p['W_fp'][C:], r(p['b_fp']),
      p['W_fpr'][:C], p['W_fpr'][C:], r(p['b_fpr']))


def kernel(image_feats, point_feats, mask, inds2d, inds3d, params):
    p = params
    flat2d = inds2d[:, 0] * P + inds2d[:, 1]
    n_ids = inds3d[:, 0]
    flat3d = n_ids * K + inds3d[:, 1]
    core_of = (jnp.arange(M, dtype=jnp.int32) // BPW) % NC
    flat3d_adj = flat3d + core_of * NK

    pub_p, pri_p, q, q_bf, kv = _point_call(point_feats[0], p)
    X, QG = _sc_gather_rows2(image_feats.reshape(V * P, C), flat2d, q, n_ids)
    pub, pri, vr, s3 = _rows_call(X, QG, p)
    sd = _sc_scatter_scores(s3.reshape(M), flat3d_adj)
    attn, aux = _softmax_call(sd.reshape(NC, N, K), q, kv)
    a_elem = _sc_gather_flat(attn.reshape(NK), flat3d)
    parts = _seg_call(pub, pri, vr, a_elem.reshape(M // 1024, 1, 1024),
                      n_ids.reshape(M // 1024, 1, 1024))
    feat, poolpub, poolpri = _final_call(parts, pub_p, pri_p, aux, kv, p)
    return feat, pub_p, pri_p, poolpub, poolpri
